# baseline (speedup 1.0000x reference)
import functools

import jax
import jax.numpy as jnp
from jax import lax
from jax.experimental import pallas as pl
from jax.experimental.pallas import tpu as pltpu
from jax.experimental.pallas import tpu_sc as plsc

_CHUNK = 128


@functools.cache
def _build(B, V, D, idx_dtype, tbl_dtype):
    info = plsc.get_sparse_core_info()
    NC, NS = info.num_cores, info.num_subcores
    NW = NC * NS
    assert B % NW == 0, (B, NW)
    b_per_w = B // NW
    n_chunks = -(-b_per_w // _CHUNK)
    assert b_per_w % _CHUNK == 0, (b_per_w, _CHUNK)

    mesh = plsc.VectorSubcoreMesh(core_axis_name="c", subcore_axis_name="s")

    @functools.partial(
        pl.kernel,
        mesh=mesh,
        out_type=jax.ShapeDtypeStruct((B, D), tbl_dtype),
        compiler_params=pltpu.CompilerParams(use_tc_tiling_on_sc=False),
        scratch_types=[
            pltpu.VMEM((b_per_w,), idx_dtype),
            pltpu.VMEM((b_per_w, D), tbl_dtype),
            pltpu.SemaphoreType.DMA,
        ],
    )
    def gather_kernel(labels_hbm, table_hbm, out_hbm, idx_v, rows_v, sem):
        wid = lax.axis_index("s") * NC + lax.axis_index("c")
        base = wid * b_per_w
        pltpu.sync_copy(labels_hbm.at[pl.ds(base, b_per_w)], idx_v)
        copies = [
            pltpu.async_copy(
                table_hbm.at[idx_v.at[pl.ds(j * _CHUNK, _CHUNK)]],
                rows_v.at[pl.ds(j * _CHUNK, _CHUNK)],
                sem,
            )
            for j in range(n_chunks)
        ]
        for c in copies:
            c.wait()
        pltpu.sync_copy(rows_v, out_hbm.at[pl.ds(base, b_per_w)])

    return gather_kernel


def kernel(labels, embed_table):
    B, = labels.shape
    V, D = embed_table.shape
    fn = _build(B, V, D, labels.dtype, embed_table.dtype)
    return fn(labels, embed_table)


# speedup vs baseline: 4.0352x; 4.0352x over previous
import functools

import jax
import jax.numpy as jnp
from jax import lax
from jax.experimental import pallas as pl
from jax.experimental.pallas import tpu as pltpu
from jax.experimental.pallas import tpu_sc as plsc

_LANE = 16
_CHUNK = 1024
_WAVE = 64


@functools.cache
def _build(B, V, D):
    info = plsc.get_sparse_core_info()
    NC, NS = info.num_cores, info.num_subcores
    NW = NC * NS
    assert D == 32 and B % (16 * NW) == 0
    n_tc = (V + 127) // 128
    n_full = n_tc // 8
    tail_tc = n_tc - n_full * 8
    base_ch, extra_ch = divmod(n_full, NW)
    n_vec = B // _LANE
    trash = B + 64

    mesh = plsc.VectorSubcoreMesh(core_axis_name="c", subcore_axis_name="s")

    @functools.partial(
        pl.kernel,
        mesh=mesh,
        out_type=jax.ShapeDtypeStruct((B * D,), jnp.float32),
        compiler_params=pltpu.CompilerParams(
            use_tc_tiling_on_sc=True,
            disable_bounds_checks=True,
            needs_layout_passes=False,
        ),
        scratch_types=[
            pltpu.VMEM((B,), jnp.int32),
            pltpu.VMEM((B + 128,), jnp.int32),
            pltpu.VMEM((B + 128,), jnp.int32),
            pltpu.VMEM((D, _CHUNK), jnp.float32),
            pltpu.VMEM((D, _CHUNK), jnp.float32),
            pltpu.VMEM((_WAVE * D,), jnp.float32),
            pltpu.SemaphoreType.DMA,
            pltpu.SemaphoreType.DMA,
            pltpu.SemaphoreType.DMA,
        ],
    )
    def scan_gather(labels_hbm, table_hbm, out_hbm, labels_v, match_i,
                    cm_i, slab_a, slab_b, colstage_v, sem_a, sem_b,
                    sem_out):
        wid = lax.axis_index("s") * NC + lax.axis_index("c")
        is_last = wid == NW - 1
        c0 = wid * base_ch + jnp.minimum(wid, extra_ch)
        nch_full = base_ch + jnp.where(wid < extra_ch, 1, 0)
        nch = nch_full + jnp.where(is_last, 1, 0)
        lane_lo = c0 * _CHUNK
        lane_hi = (c0 + nch_full) * _CHUNK + jnp.where(
            is_last, tail_tc * 128, 0
        )

        pltpu.sync_copy(labels_hbm, labels_v)
        c_iota = lax.broadcasted_iota(jnp.int32, (_LANE,), 0)

        def slab_copies(ch, slab_v, sem, tail):
            w = tail_tc * 128 if tail else _CHUNK
            l0 = pl.multiple_of((c0 + ch) * _CHUNK, 128)
            return [
                pltpu.make_async_copy(
                    table_hbm.at[pl.ds(8 * ci, 8), pl.ds(l0, w)],
                    slab_v.at[pl.ds(8 * ci, 8), pl.ds(0, w)],
                    sem,
                )
                for ci in range(4)
            ]

        def fire(ch, slab_v, sem):
            is_tail = is_last & (ch == nch - 1)

            @pl.when(~is_tail)
            def _():
                for cp in slab_copies(ch, slab_v, sem, False):
                    cp.start()

            @pl.when(is_tail)
            def _():
                for cp in slab_copies(ch, slab_v, sem, True):
                    cp.start()

        def wait(ch, slab_v, sem):
            is_tail = is_last & (ch == nch - 1)

            @pl.when(~is_tail)
            def _():
                for cp in slab_copies(ch, slab_v, sem, False):
                    cp.wait()

            @pl.when(is_tail)
            def _():
                for cp in slab_copies(ch, slab_v, sem, True):
                    cp.wait()

        def bucket(v, off):
            r = labels_v[pl.ds(v * _LANE, _LANE)]
            i = c_iota + v * _LANE
            m = (r >= lane_lo) & (r < lane_hi)
            mi = m.astype(jnp.int32)
            incl = jnp.cumsum(mi)
            tgt = jnp.where(m, off + incl - 1, trash)
            plsc.store_scatter(match_i, [tgt], i)
            return off + incl[_LANE - 1]

        fire(0, slab_a, sem_a)
        nm = lax.fori_loop(0, n_vec, bucket, 0, unroll=4)
        nmv = (nm + _LANE - 1) // _LANE

        def process(ch, slab_v):
            is_tail = is_last & (ch == nch - 1)
            l0 = pl.multiple_of((c0 + ch) * _CHUNK, 128)
            width = jnp.where(is_tail, tail_tc * 128, _CHUNK)

            def rescan(u, off):
                ii = match_i[pl.ds(u * _LANE, _LANE)]
                rr = plsc.load_gather(labels_v, [ii & (B - 1)])
                valid = (u * _LANE + c_iota) < nm
                m = valid & (rr >= l0) & (rr < l0 + width)
                mi = m.astype(jnp.int32)
                incl = jnp.cumsum(mi)
                tgt = jnp.where(m, off + incl - 1, trash)
                plsc.store_scatter(cm_i, [tgt], ii)
                return off + incl[_LANE - 1]

            cm_n = lax.fori_loop(0, nmv, rescan, 0, unroll=False)

            def wave(wv, _):
                k0 = pl.multiple_of(wv * _WAVE, _WAVE)
                kn = jnp.minimum(_WAVE, cm_n - k0)
                for g in range(_WAVE // _LANE):

                    @pl.when(g * _LANE < kn)
                    def _():
                        kvec = c_iota + g * _LANE
                        present = kvec < kn
                        ivec = cm_i[pl.ds(k0 + g * _LANE, _LANE)]
                        ivec = jnp.where(present, ivec, 0)
                        rr = plsc.load_gather(labels_v, [ivec])
                        rloc = jnp.where(present, rr - l0, 0)
                        for c in range(D):
                            vals = plsc.load_gather(
                                slab_v,
                                [jnp.full((_LANE,), c, jnp.int32), rloc],
                            )
                            plsc.store_scatter(
                                colstage_v, [kvec * D + c], vals
                            )
                        for lane in range(_LANE):
                            k = g * _LANE + lane

                            @pl.when(k < kn)
                            def _():
                                row = pl.multiple_of(ivec[lane] * D, 8)
                                pltpu.async_copy(
                                    colstage_v.at[pl.ds(k * D, D)],
                                    out_hbm.at[pl.ds(row, D)],
                                    sem_out,
                                )

                def drain(k, _):
                    pltpu.make_async_copy(
                        colstage_v.at[pl.ds(0, D)],
                        out_hbm.at[pl.ds(0, D)],
                        sem_out,
                    ).wait()
                    return 0

                lax.fori_loop(0, kn, drain, 0, unroll=False)
                return 0

            lax.fori_loop(0, (cm_n + _WAVE - 1) // _WAVE, wave, 0,
                          unroll=False)

        def outer(oc, _):
            ch_a = oc * 2
            ch_b = oc * 2 + 1

            @pl.when(ch_a < nch)
            def _():
                @pl.when(ch_b < nch)
                def _():
                    fire(ch_b, slab_b, sem_b)

                wait(ch_a, slab_a, sem_a)
                process(ch_a, slab_a)

            @pl.when(ch_b < nch)
            def _():
                @pl.when(ch_b + 1 < nch)
                def _():
                    fire(ch_b + 1, slab_a, sem_a)

                wait(ch_b, slab_b, sem_b)
                process(ch_b, slab_b)

            return 0

        lax.fori_loop(0, (nch + 1) // 2, outer, 0, unroll=False)

    return scan_gather


def kernel(labels, embed_table):
    B, = labels.shape
    V, D = embed_table.shape
    fn = _build(B, V, D)
    out_flat = fn(labels, embed_table.T)
    return out_flat.reshape(B, D)
